# Initial kernel scaffold; baseline (speedup 1.0000x reference)
import jax
import jax.numpy as jnp
from jax.experimental import pallas as pl


def kernel(x, senders, receivers, edges, globals_, np_params, theta, W, b, ln_scale, ln_bias, Wd, bd):
    raise NotImplementedError("write your pallas kernel here")


# speedup vs baseline: 44.9063x; 44.9063x over previous
import functools

import jax
import jax.numpy as jnp
from jax import lax
from jax.experimental import pallas as pl
from jax.experimental.pallas import tpu as pltpu
from jax.experimental.pallas import tpu_sc as plsc

N = 100000
E = 6400000
D = 4
NPAD = 102400
NCORES = 2
NSUB = 16
NW = NCORES * NSUB
EP_ROWS = 50688
E_PAD = EP_ROWS * 128
ROWS_PER_W = EP_ROWS // NW
KC = 24
NCHUNK = ROWS_PER_W // KC
TROWS = NPAD // NSUB
T128 = NPAD * D // 128
ST_ROWS = T128 // NSUB
PPIECE = 8
P128 = ST_ROWS // PPIECE
P4 = P128 * 128 // D
CONV_IT = P128 * 128 // 16
BLK = 1024
NBLK = NPAD // BLK


def _mesh():
    return plsc.VectorSubcoreMesh(core_axis_name="c", subcore_axis_name="s",
                                  num_cores=NCORES, num_subcores=NSUB)


def _iota16():
    return jnp.arange(16, dtype=jnp.int32)


def _fill_t4(t4, val):
    v = jnp.full((16,), val, jnp.float32)
    it = _iota16()

    def body(k, carry):
        w = k * 16
        plsc.store_scatter(t4, [(w + it) >> 2, (w + it) & 3], v)
        return carry

    lax.fori_loop(0, CONV_IT, body, 0, unroll=8)


def _conv_128_to_4(st128, t4):
    it = _iota16()

    def body(k, carry):
        w = k * 16
        v = st128[w >> 7, pl.ds((w & 127), 16)]
        plsc.store_scatter(t4, [(w + it) >> 2, (w + it) & 3], v)
        return carry

    lax.fori_loop(0, CONV_IT, body, 0, unroll=8)


def _conv_4_to_128(t4, st128):
    it = _iota16()

    def body(k, carry):
        w = k * 16
        v = plsc.load_gather(t4, [(w + it) >> 2, (w + it) & 3])
        st128[w >> 7, pl.ds((w & 127), 16)] = v
        return carry

    lax.fori_loop(0, CONV_IT, body, 0, unroll=8)


def _spmem_to_hbm128(acc, t4, st128, nbase, obase, out_ref):
    for p in range(PPIECE):
        pltpu.sync_copy(acc.at[pl.ds(nbase + p * P4, P4)], t4)
        _conv_4_to_128(t4, st128)
        pltpu.sync_copy(st128, out_ref.at[pl.ds(obase + p * P128, P128)])


NPAIR = NCHUNK // 2


def _deg_body(s2d, r2d, out_s, out_r, acc_s, acc_r, ones_v, t4, st128,
              sidxA, ridxA, sidxB, ridxB, ssemA, ssemB):
    c = lax.axis_index("c")
    s = lax.axis_index("s")
    w = c * NSUB + s
    it = _iota16()
    onev = jnp.ones((16,), jnp.float32)
    for k in range(32):
        plsc.store_scatter(ones_v, [(k * 16 + it) >> 2, (k * 16 + it) & 3], onev)
    _fill_t4(t4, 0.0)
    for p in range(PPIECE):
        psl = pl.ds(s * TROWS + p * P4, P4)
        pltpu.sync_copy(t4, acc_s.at[psl])
        pltpu.sync_copy(t4, acc_r.at[psl])
    plsc.subcore_barrier()
    base = w * ROWS_PER_W

    def fire(sidx, ridx, sem):
        return ([pltpu.async_copy(ones_v, acc_s.at[sidx.at[j]], sem, add=True)
                 for j in range(KC)]
                + [pltpu.async_copy(ones_v, acc_r.at[ridx.at[j]], sem, add=True)
                   for j in range(KC)])

    def pair(p, carry):
        rowA = base + (2 * p) * KC
        rowB = rowA + KC
        pltpu.sync_copy(s2d.at[pl.ds(rowA, KC)], sidxA)
        pltpu.sync_copy(r2d.at[pl.ds(rowA, KC)], ridxA)
        sdA = fire(sidxA, ridxA, ssemA)
        pltpu.sync_copy(s2d.at[pl.ds(rowB, KC)], sidxB)
        pltpu.sync_copy(r2d.at[pl.ds(rowB, KC)], ridxB)
        sdB = fire(sidxB, ridxB, ssemB)
        for d in sdA:
            d.wait()
        for d in sdB:
            d.wait()
        return carry

    lax.fori_loop(0, NPAIR, pair, 0)
    plsc.subcore_barrier()
    _spmem_to_hbm128(acc_s, t4, st128, s * TROWS, s * ST_ROWS, out_s.at[c])
    _spmem_to_hbm128(acc_r, t4, st128, s * TROWS, s * ST_ROWS, out_r.at[c])


@jax.jit
def _sc_degrees(s2d, r2d):
    k = pl.kernel(
        _deg_body,
        out_type=[
            jax.ShapeDtypeStruct((NCORES, T128, 128), jnp.float32),
            jax.ShapeDtypeStruct((NCORES, T128, 128), jnp.float32),
        ],
        mesh=_mesh(),
        compiler_params=pltpu.CompilerParams(use_tc_tiling_on_sc=False,
                                            needs_layout_passes=False),
        scratch_types=[
            pltpu.VMEM_SHARED((NPAD, D), jnp.float32),
            pltpu.VMEM_SHARED((NPAD, D), jnp.float32),
            pltpu.VMEM((128, D), jnp.float32),
            pltpu.VMEM((P4, D), jnp.float32),
            pltpu.VMEM((P128, 128), jnp.float32),
            pltpu.VMEM((KC, 128), jnp.int32),
            pltpu.VMEM((KC, 128), jnp.int32),
            pltpu.VMEM((KC, 128), jnp.int32),
            pltpu.VMEM((KC, 128), jnp.int32),
            pltpu.SemaphoreType.DMA,
            pltpu.SemaphoreType.DMA,
        ],
    )
    return k(s2d, r2d)


def _msg_body(s2d, r2d, table128, out, tbh, tb4, acc, t4, st128,
              sidxA, ridxA, sidxB, ridxB, rowsA, rowsB, gsem, ssemA, ssemB):
    c = lax.axis_index("c")
    s = lax.axis_index("s")
    w = c * NSUB + s
    _fill_t4(t4, 0.0)
    for p in range(PPIECE):
        pltpu.sync_copy(t4, acc.at[pl.ds(s * TROWS + p * P4, P4)])
    for p in range(PPIECE):
        pltpu.sync_copy(table128.at[pl.ds(s * ST_ROWS + p * P128, P128)], st128)
        _conv_128_to_4(st128, t4)
        pltpu.sync_copy(t4, tb4.at[pl.ds(s * TROWS + p * P4, P4)])
        pltpu.sync_copy(t4, tbh.at[c, pl.ds(s * TROWS + p * P4, P4)])
    plsc.subcore_barrier()
    base = w * ROWS_PER_W

    def pair(p, carry):
        rowA = base + (2 * p) * KC
        rowB = rowA + KC
        pltpu.sync_copy(s2d.at[pl.ds(rowA, KC)], sidxA)
        pltpu.sync_copy(r2d.at[pl.ds(rowA, KC)], ridxA)
        gdA = [pltpu.async_copy(tb4.at[sidxA.at[j]], rowsA.at[j], gsem)
               for j in range(KC)]
        pltpu.sync_copy(s2d.at[pl.ds(rowB, KC)], sidxB)
        pltpu.sync_copy(r2d.at[pl.ds(rowB, KC)], ridxB)
        for d in gdA:
            d.wait()
        sdA = [pltpu.async_copy(rowsA.at[j], acc.at[ridxA.at[j]], ssemA,
                                add=True) for j in range(KC)]
        gdB = [pltpu.async_copy(tbh.at[c].at[sidxB.at[j]], rowsB.at[j], gsem)
               for j in range(KC)]
        for d in gdB:
            d.wait()
        sdB = [pltpu.async_copy(rowsB.at[j], acc.at[ridxB.at[j]], ssemB,
                                add=True) for j in range(KC)]
        for d in sdA:
            d.wait()
        for d in sdB:
            d.wait()
        return carry

    lax.fori_loop(0, NPAIR, pair, 0)
    plsc.subcore_barrier()
    _spmem_to_hbm128(acc, t4, st128, s * TROWS, s * ST_ROWS, out.at[c])


@jax.jit
def _sc_messages(s2d, r2d, table128):
    k = pl.kernel(
        _msg_body,
        out_type=jax.ShapeDtypeStruct((NCORES, T128, 128), jnp.float32),
        mesh=_mesh(),
        compiler_params=pltpu.CompilerParams(use_tc_tiling_on_sc=False,
                                            needs_layout_passes=False),
        scratch_types=[
            pltpu.HBM((NCORES, NPAD, D), jnp.float32),
            pltpu.VMEM_SHARED((NPAD, D), jnp.float32),
            pltpu.VMEM_SHARED((NPAD, D), jnp.float32),
            pltpu.VMEM((P4, D), jnp.float32),
            pltpu.VMEM((P128, 128), jnp.float32),
            pltpu.VMEM((KC, 128), jnp.int32),
            pltpu.VMEM((KC, 128), jnp.int32),
            pltpu.VMEM((KC, 128), jnp.int32),
            pltpu.VMEM((KC, 128), jnp.int32),
            pltpu.VMEM((KC, 128, D), jnp.float32),
            pltpu.VMEM((KC, 128, D), jnp.float32),
            pltpu.SemaphoreType.DMA,
            pltpu.SemaphoreType.DMA,
            pltpu.SemaphoreType.DMA,
        ],
    )
    return k(s2d, r2d, table128)


def _embed_k(x_ref, p_ref, o_ref):
    x = x_ref[...][:, :4]
    o_ref[...] = jnp.cos(p_ref[0:1, :4] * x + p_ref[1:2, :4])


def _table_k(n_ref, ds0_ref, ds1_ref, w0_ref, w1_ref, pp_ref, o_ref):
    h = n_ref[...]
    for j, wr in enumerate((w0_ref, w1_ref)):
        lin = jnp.dot(h, wr[...], preferred_element_type=jnp.float32,
                      precision=lax.Precision.HIGHEST) + pp_ref[j:j + 1, :]
        t = jnp.tanh(lin)
        h = jnp.cos(pp_ref[2 + 2 * j:3 + 2 * j, :] * t
                    + pp_ref[3 + 2 * j:4 + 2 * j, :])
    inv = lax.rsqrt(ds0_ref[...] + ds1_ref[...] + 1.0)
    o_ref[...] = h * inv


def _combine_k(p0_ref, p1_ref, t_ref, n_ref, dr0_ref, dr1_ref, mk_ref,
               lp_ref, o_ref):
    inv = lax.rsqrt(dr0_ref[...] + dr1_ref[...] + 1.0)
    v = (p0_ref[...] + p1_ref[...] + t_ref[...]) * inv + n_ref[...]
    mk = mk_ref[...]
    mu = jnp.dot(v, mk, preferred_element_type=jnp.float32,
                 precision=lax.Precision.HIGHEST)
    d = v - mu
    var = jnp.dot(d * d, mk, preferred_element_type=jnp.float32,
                  precision=lax.Precision.HIGHEST)
    o_ref[...] = d * lax.rsqrt(var + 1e-6) * lp_ref[0:1, :] + lp_ref[1:2, :]


def _pool_k(n_ref, o_ref):
    i = pl.program_id(0)
    rows = lax.broadcasted_iota(jnp.int32, (PBLK, 1), 0) + i * PBLK
    m = (rows < NROWS128).astype(jnp.float32)
    ssum = jnp.sum(n_ref[...] * m, axis=0, keepdims=True)

    @pl.when(i == 0)
    def _():
        o_ref[...] = jnp.zeros_like(o_ref)

    o_ref[0:1, :] = o_ref[0:1, :] + ssum


def _edges_k(e_ref, o_ref):
    o_ref[...] = e_ref[...] * 4.0


NBLK128 = 8
B128 = T128 // NBLK128
_t128_spec = pl.BlockSpec((B128, 128), lambda i: (i, 0))
_kron_spec = pl.BlockSpec((128, 128), lambda i: (0, 0))
_row_spec = pl.BlockSpec((8, 128), lambda i: (0, 0))

XBLK = 800


def _tc_embed():
    return pl.pallas_call(
        _embed_k,
        grid=(NPAD // XBLK,),
        in_specs=[pl.BlockSpec((XBLK, 128), lambda i: (jnp.minimum(i, N // XBLK - 1), 0)),
                  pl.BlockSpec((8, D), lambda i: (0, 0))],
        out_specs=pl.BlockSpec((XBLK, D), lambda i: (i, 0)),
        out_shape=jax.ShapeDtypeStruct((NPAD, D), jnp.float32),
    )


def _tc_table():
    return pl.pallas_call(
        _table_k,
        grid=(NBLK128,),
        in_specs=[_t128_spec, _t128_spec, _t128_spec,
                  _kron_spec, _kron_spec, _row_spec],
        out_specs=_t128_spec,
        out_shape=jax.ShapeDtypeStruct((T128, 128), jnp.float32),
    )


def _tc_combine():
    return pl.pallas_call(
        _combine_k,
        grid=(NBLK128,),
        in_specs=[_t128_spec] * 6 + [_kron_spec, _row_spec],
        out_specs=_t128_spec,
        out_shape=jax.ShapeDtypeStruct((T128, 128), jnp.float32),
    )


NROWS128 = N * D // 128
PBLK = 400


def _tc_pool():
    return pl.pallas_call(
        _pool_k,
        grid=(T128 // PBLK,),
        in_specs=[pl.BlockSpec((PBLK, 128), lambda i: (i, 0))],
        out_specs=pl.BlockSpec((8, 128), lambda i: (0, 0)),
        out_shape=jax.ShapeDtypeStruct((8, 128), jnp.float32),
    )


EROWS2 = E * D // 128
EBLK = 2000


def _tc_edges():
    return pl.pallas_call(
        _edges_k,
        grid=(EROWS2 // EBLK,),
        in_specs=[pl.BlockSpec((EBLK, 128), lambda i: (i, 0))],
        out_specs=pl.BlockSpec((EBLK, 128), lambda i: (i, 0)),
        out_shape=jax.ShapeDtypeStruct((EROWS2, 128), jnp.float32),
    )


def kernel(x, senders, receivers, edges, globals_, np_params, theta, W, b,
           ln_scale, ln_bias, Wd, bd):
    f32 = jnp.float32
    pad_idx = jnp.full((E_PAD - E,), N, jnp.int32)
    s2d = jnp.concatenate([senders.astype(jnp.int32), pad_idx]).reshape(EP_ROWS, 128)
    r2d = jnp.concatenate([receivers.astype(jnp.int32), pad_idx]).reshape(EP_ROWS, 128)
    np_pack = jnp.concatenate([np_params.astype(f32),
                               jnp.zeros((6, D), f32)], axis=0)
    eye32 = jnp.eye(32, dtype=f32)
    mk = jnp.kron(eye32, jnp.full((4, 4), 0.25, f32))

    nodes = _tc_embed()(x, np_pack).reshape(T128, 128)

    deg_s, deg_r = _sc_degrees(s2d, r2d)

    for step in range(2):
        w0 = jnp.kron(eye32, W[step, 0, :, :4].astype(f32))
        w1 = jnp.kron(eye32, W[step, 1, :, :4].astype(f32))
        pp = jnp.tile(jnp.concatenate([b[step, :, :4],
                                       theta[step].reshape(4, 4),
                                       jnp.zeros((2, 4), f32)],
                                      axis=0).astype(f32), (1, 32))
        lp = jnp.tile(jnp.concatenate([ln_scale[step][None, :],
                                       ln_bias[step][None, :],
                                       jnp.zeros((6, 4), f32)],
                                      axis=0).astype(f32), (1, 32))
        table = _tc_table()(nodes, deg_s[0], deg_s[1], w0, w1, pp)
        partials = _sc_messages(s2d, r2d, table)
        nodes = _tc_combine()(partials[0], partials[1], table, nodes,
                              deg_r[0], deg_r[1], mk, lp)

    pool = _tc_pool()(nodes)
    pooled = pool[0:1, :].reshape(32, 4).sum(axis=0, keepdims=True) / float(N)
    globals_out = pooled @ Wd + bd[None, :]

    e128 = (edges.reshape(E // 128, 128, D)
            .transpose(0, 2, 1).reshape(EROWS2, 128))
    o128 = _tc_edges()(e128)
    edges_out = (o128.reshape(E // 128, D, 128)
                 .transpose(0, 2, 1).reshape(E, D))
    nodes_out = nodes.reshape(NPAD, D)[:N]
    return nodes_out, edges_out, globals_out


# speedup vs baseline: 64.6697x; 1.4401x over previous
import functools

import jax
import jax.numpy as jnp
from jax import lax
from jax.experimental import pallas as pl
from jax.experimental.pallas import tpu as pltpu
from jax.experimental.pallas import tpu_sc as plsc

N = 100000
E = 6400000
D = 4
NPAD = 102400
NCORES = 2
NSUB = 16
NW = NCORES * NSUB
EP_ROWS = 50688
E_PAD = EP_ROWS * 128
ROWS_PER_W = EP_ROWS // NW
KC = 16
NCHUNK = ROWS_PER_W // KC
TROWS = NPAD // NSUB
T128 = NPAD * D // 128
ST_ROWS = T128 // NSUB
PPIECE = 8
P128 = ST_ROWS // PPIECE
P4 = P128 * 128 // D
CONV_IT = P128 * 128 // 16
BLK = 1024
NBLK = NPAD // BLK


def _mesh():
    return plsc.VectorSubcoreMesh(core_axis_name="c", subcore_axis_name="s",
                                  num_cores=NCORES, num_subcores=NSUB)


def _iota16():
    return jnp.arange(16, dtype=jnp.int32)


def _fill_t4(t4, val):
    v = jnp.full((16,), val, jnp.float32)
    it = _iota16()

    def body(k, carry):
        w = k * 16
        plsc.store_scatter(t4, [(w + it) >> 2, (w + it) & 3], v)
        return carry

    lax.fori_loop(0, CONV_IT, body, 0, unroll=8)


def _conv_128_to_4(st128, t4):
    it = _iota16()

    def body(k, carry):
        w = k * 16
        v = st128[w >> 7, pl.ds((w & 127), 16)]
        plsc.store_scatter(t4, [(w + it) >> 2, (w + it) & 3], v)
        return carry

    lax.fori_loop(0, CONV_IT, body, 0, unroll=8)


def _conv_4_to_128(t4, st128):
    it = _iota16()

    def body(k, carry):
        w = k * 16
        v = plsc.load_gather(t4, [(w + it) >> 2, (w + it) & 3])
        st128[w >> 7, pl.ds((w & 127), 16)] = v
        return carry

    lax.fori_loop(0, CONV_IT, body, 0, unroll=8)


def _spmem_to_hbm128(acc, t4, st128, nbase, obase, out_ref):
    for p in range(PPIECE):
        pltpu.sync_copy(acc.at[pl.ds(nbase + p * P4, P4)], t4)
        _conv_4_to_128(t4, st128)
        pltpu.sync_copy(st128, out_ref.at[pl.ds(obase + p * P128, P128)])


NPAIR = NCHUNK // 2
NGRP = NCHUNK // 3


def _deg_body(s2d, r2d, out_s, out_r, acc_s, acc_r, ones_v, t4, st128,
              sidxA, ridxA, sidxB, ridxB, sidxC, ridxC,
              ssemA, ssemB, ssemC, isem2, isem3):
    c = lax.axis_index("c")
    s = lax.axis_index("s")
    w = c * NSUB + s
    it = _iota16()
    onev = jnp.ones((16,), jnp.float32)
    for k in range(32):
        plsc.store_scatter(ones_v, [(k * 16 + it) >> 2, (k * 16 + it) & 3], onev)
    _fill_t4(t4, 0.0)
    for p in range(PPIECE):
        psl = pl.ds(s * TROWS + p * P4, P4)
        pltpu.sync_copy(t4, acc_s.at[psl])
        pltpu.sync_copy(t4, acc_r.at[psl])
    plsc.subcore_barrier()
    base = w * ROWS_PER_W

    def fire(sidx, ridx, sem):
        return ([pltpu.async_copy(ones_v, acc_s.at[sidx.at[j]], sem, add=True)
                 for j in range(KC)]
                + [pltpu.async_copy(ones_v, acc_r.at[ridx.at[j]], sem, add=True)
                   for j in range(KC)])

    def grp(p, carry):
        rowA = base + (3 * p) * KC
        pltpu.sync_copy(s2d.at[pl.ds(rowA, KC)], sidxA)
        pltpu.sync_copy(r2d.at[pl.ds(rowA, KC)], ridxA)
        i2 = [pltpu.async_copy(s2d.at[pl.ds(rowA + KC, KC)], sidxB, isem2),
              pltpu.async_copy(r2d.at[pl.ds(rowA + KC, KC)], ridxB, isem2)]
        i3 = [pltpu.async_copy(s2d.at[pl.ds(rowA + 2 * KC, KC)], sidxC, isem3),
              pltpu.async_copy(r2d.at[pl.ds(rowA + 2 * KC, KC)], ridxC, isem3)]
        sdA = fire(sidxA, ridxA, ssemA)
        for d in i2:
            d.wait()
        sdB = fire(sidxB, ridxB, ssemB)
        for d in i3:
            d.wait()
        sdC = fire(sidxC, ridxC, ssemC)
        for d in sdA + sdB + sdC:
            d.wait()
        return carry

    lax.fori_loop(0, NGRP, grp, 0)
    plsc.subcore_barrier()
    _spmem_to_hbm128(acc_s, t4, st128, s * TROWS, s * ST_ROWS, out_s.at[c])
    _spmem_to_hbm128(acc_r, t4, st128, s * TROWS, s * ST_ROWS, out_r.at[c])


@jax.jit
def _sc_degrees(s2d, r2d):
    k = pl.kernel(
        _deg_body,
        out_type=[
            jax.ShapeDtypeStruct((NCORES, T128, 128), jnp.float32),
            jax.ShapeDtypeStruct((NCORES, T128, 128), jnp.float32),
        ],
        mesh=_mesh(),
        compiler_params=pltpu.CompilerParams(use_tc_tiling_on_sc=False,
                                            needs_layout_passes=False),
        scratch_types=[
            pltpu.VMEM_SHARED((NPAD, D), jnp.float32),
            pltpu.VMEM_SHARED((NPAD, D), jnp.float32),
            pltpu.VMEM((128, D), jnp.float32),
            pltpu.VMEM((P4, D), jnp.float32),
            pltpu.VMEM((P128, 128), jnp.float32),
            pltpu.VMEM((KC, 128), jnp.int32),
            pltpu.VMEM((KC, 128), jnp.int32),
            pltpu.VMEM((KC, 128), jnp.int32),
            pltpu.VMEM((KC, 128), jnp.int32),
            pltpu.VMEM((KC, 128), jnp.int32),
            pltpu.VMEM((KC, 128), jnp.int32),
            pltpu.SemaphoreType.DMA,
            pltpu.SemaphoreType.DMA,
            pltpu.SemaphoreType.DMA,
            pltpu.SemaphoreType.DMA,
            pltpu.SemaphoreType.DMA,
        ],
    )
    return k(s2d, r2d)


def _msg_body(s2d, r2d, table128, out, tb4, acc, t4, st128,
              sidxA, ridxA, sidxB, ridxB, sidxC, ridxC, rowsA, rowsB, rowsC,
              gsem, ssemA, ssemB, ssemC, isem2, isem3):
    c = lax.axis_index("c")
    s = lax.axis_index("s")
    w = c * NSUB + s
    _fill_t4(t4, 0.0)
    for p in range(PPIECE):
        pltpu.sync_copy(t4, acc.at[pl.ds(s * TROWS + p * P4, P4)])
    for p in range(PPIECE):
        pltpu.sync_copy(table128.at[pl.ds(s * ST_ROWS + p * P128, P128)], st128)
        _conv_128_to_4(st128, t4)
        pltpu.sync_copy(t4, tb4.at[pl.ds(s * TROWS + p * P4, P4)])
    plsc.subcore_barrier()
    base = w * ROWS_PER_W

    def grp(p, carry):
        rowA = base + (3 * p) * KC
        pltpu.sync_copy(s2d.at[pl.ds(rowA, KC)], sidxA)
        pltpu.sync_copy(r2d.at[pl.ds(rowA, KC)], ridxA)
        i2 = [pltpu.async_copy(s2d.at[pl.ds(rowA + KC, KC)], sidxB, isem2),
              pltpu.async_copy(r2d.at[pl.ds(rowA + KC, KC)], ridxB, isem2)]
        i3 = [pltpu.async_copy(s2d.at[pl.ds(rowA + 2 * KC, KC)], sidxC, isem3),
              pltpu.async_copy(r2d.at[pl.ds(rowA + 2 * KC, KC)], ridxC, isem3)]
        gdA = [pltpu.async_copy(tb4.at[sidxA.at[j]], rowsA.at[j], gsem)
               for j in range(KC)]
        for d in gdA:
            d.wait()
        sdA = [pltpu.async_copy(rowsA.at[j], acc.at[ridxA.at[j]], ssemA,
                                add=True) for j in range(KC)]
        for d in i2:
            d.wait()
        gdB = [pltpu.async_copy(tb4.at[sidxB.at[j]], rowsB.at[j], gsem)
               for j in range(KC)]
        for d in gdB:
            d.wait()
        sdB = [pltpu.async_copy(rowsB.at[j], acc.at[ridxB.at[j]], ssemB,
                                add=True) for j in range(KC)]
        for d in i3:
            d.wait()
        gdC = [pltpu.async_copy(tb4.at[sidxC.at[j]], rowsC.at[j], gsem)
               for j in range(KC)]
        for d in gdC:
            d.wait()
        sdC = [pltpu.async_copy(rowsC.at[j], acc.at[ridxC.at[j]], ssemC,
                                add=True) for j in range(KC)]
        for d in sdA + sdB + sdC:
            d.wait()
        return carry

    lax.fori_loop(0, NGRP, grp, 0)
    plsc.subcore_barrier()
    _spmem_to_hbm128(acc, t4, st128, s * TROWS, s * ST_ROWS, out.at[c])


@jax.jit
def _sc_messages(s2d, r2d, table128):
    k = pl.kernel(
        _msg_body,
        out_type=jax.ShapeDtypeStruct((NCORES, T128, 128), jnp.float32),
        mesh=_mesh(),
        compiler_params=pltpu.CompilerParams(use_tc_tiling_on_sc=False,
                                            needs_layout_passes=False),
        scratch_types=[
            pltpu.VMEM_SHARED((NPAD, D), jnp.float32),
            pltpu.VMEM_SHARED((NPAD, D), jnp.float32),
            pltpu.VMEM((P4, D), jnp.float32),
            pltpu.VMEM((P128, 128), jnp.float32),
            pltpu.VMEM((KC, 128), jnp.int32),
            pltpu.VMEM((KC, 128), jnp.int32),
            pltpu.VMEM((KC, 128), jnp.int32),
            pltpu.VMEM((KC, 128), jnp.int32),
            pltpu.VMEM((KC, 128), jnp.int32),
            pltpu.VMEM((KC, 128), jnp.int32),
            pltpu.VMEM((KC, 128, D), jnp.float32),
            pltpu.VMEM((KC, 128, D), jnp.float32),
            pltpu.VMEM((KC, 128, D), jnp.float32),
            pltpu.SemaphoreType.DMA,
            pltpu.SemaphoreType.DMA,
            pltpu.SemaphoreType.DMA,
            pltpu.SemaphoreType.DMA,
            pltpu.SemaphoreType.DMA,
            pltpu.SemaphoreType.DMA,
        ],
    )
    return k(s2d, r2d, table128)


def _embed_k(x_ref, p_ref, o_ref):
    x = x_ref[...][:, :4]
    o_ref[...] = jnp.cos(p_ref[0:1, :4] * x + p_ref[1:2, :4])


def _table_k(n_ref, ds0_ref, ds1_ref, w0_ref, w1_ref, pp_ref, o_ref):
    h = n_ref[...]
    for j, wr in enumerate((w0_ref, w1_ref)):
        lin = jnp.dot(h, wr[...], preferred_element_type=jnp.float32,
                      precision=lax.Precision.HIGHEST) + pp_ref[j:j + 1, :]
        t = jnp.tanh(lin)
        h = jnp.cos(pp_ref[2 + 2 * j:3 + 2 * j, :] * t
                    + pp_ref[3 + 2 * j:4 + 2 * j, :])
    inv = lax.rsqrt(ds0_ref[...] + ds1_ref[...] + 1.0)
    o_ref[...] = h * inv


def _combine_k(p0_ref, p1_ref, t_ref, n_ref, dr0_ref, dr1_ref, mk_ref,
               lp_ref, o_ref):
    inv = lax.rsqrt(dr0_ref[...] + dr1_ref[...] + 1.0)
    v = (p0_ref[...] + p1_ref[...] + t_ref[...]) * inv + n_ref[...]
    mk = mk_ref[...]
    mu = jnp.dot(v, mk, preferred_element_type=jnp.float32,
                 precision=lax.Precision.HIGHEST)
    d = v - mu
    var = jnp.dot(d * d, mk, preferred_element_type=jnp.float32,
                  precision=lax.Precision.HIGHEST)
    o_ref[...] = d * lax.rsqrt(var + 1e-6) * lp_ref[0:1, :] + lp_ref[1:2, :]


def _pool_k(n_ref, o_ref):
    i = pl.program_id(0)
    rows = lax.broadcasted_iota(jnp.int32, (PBLK, 1), 0) + i * PBLK
    m = (rows < NROWS128).astype(jnp.float32)
    ssum = jnp.sum(n_ref[...] * m, axis=0, keepdims=True)

    @pl.when(i == 0)
    def _():
        o_ref[...] = jnp.zeros_like(o_ref)

    o_ref[0:1, :] = o_ref[0:1, :] + ssum


def _edges_k(e_ref, o_ref):
    o_ref[...] = e_ref[...] * 4.0


NBLK128 = 8
B128 = T128 // NBLK128
_t128_spec = pl.BlockSpec((B128, 128), lambda i: (i, 0))
_kron_spec = pl.BlockSpec((128, 128), lambda i: (0, 0))
_row_spec = pl.BlockSpec((8, 128), lambda i: (0, 0))

XBLK = 800


def _tc_embed():
    return pl.pallas_call(
        _embed_k,
        grid=(NPAD // XBLK,),
        in_specs=[pl.BlockSpec((XBLK, 128), lambda i: (jnp.minimum(i, N // XBLK - 1), 0)),
                  pl.BlockSpec((8, D), lambda i: (0, 0))],
        out_specs=pl.BlockSpec((XBLK, D), lambda i: (i, 0)),
        out_shape=jax.ShapeDtypeStruct((NPAD, D), jnp.float32),
    )


def _tc_table():
    return pl.pallas_call(
        _table_k,
        grid=(NBLK128,),
        in_specs=[_t128_spec, _t128_spec, _t128_spec,
                  _kron_spec, _kron_spec, _row_spec],
        out_specs=_t128_spec,
        out_shape=jax.ShapeDtypeStruct((T128, 128), jnp.float32),
    )


def _tc_combine():
    return pl.pallas_call(
        _combine_k,
        grid=(NBLK128,),
        in_specs=[_t128_spec] * 6 + [_kron_spec, _row_spec],
        out_specs=_t128_spec,
        out_shape=jax.ShapeDtypeStruct((T128, 128), jnp.float32),
    )


NROWS128 = N * D // 128
PBLK = 400


def _tc_pool():
    return pl.pallas_call(
        _pool_k,
        grid=(T128 // PBLK,),
        in_specs=[pl.BlockSpec((PBLK, 128), lambda i: (i, 0))],
        out_specs=pl.BlockSpec((8, 128), lambda i: (0, 0)),
        out_shape=jax.ShapeDtypeStruct((8, 128), jnp.float32),
    )


EROWS2 = E * D // 128
EBLK = 2000


def _tc_edges():
    return pl.pallas_call(
        _edges_k,
        grid=(EROWS2 // EBLK,),
        in_specs=[pl.BlockSpec((EBLK, 128), lambda i: (i, 0))],
        out_specs=pl.BlockSpec((EBLK, 128), lambda i: (i, 0)),
        out_shape=jax.ShapeDtypeStruct((EROWS2, 128), jnp.float32),
    )


def kernel(x, senders, receivers, edges, globals_, np_params, theta, W, b,
           ln_scale, ln_bias, Wd, bd):
    f32 = jnp.float32
    pad_idx = jnp.full((E_PAD - E,), N, jnp.int32)
    s2d = jnp.concatenate([senders.astype(jnp.int32), pad_idx]).reshape(EP_ROWS, 128)
    r2d = jnp.concatenate([receivers.astype(jnp.int32), pad_idx]).reshape(EP_ROWS, 128)
    np_pack = jnp.concatenate([np_params.astype(f32),
                               jnp.zeros((6, D), f32)], axis=0)
    eye32 = jnp.eye(32, dtype=f32)
    mk = jnp.kron(eye32, jnp.full((4, 4), 0.25, f32))

    nodes = _tc_embed()(x, np_pack).reshape(T128, 128)

    deg_s, deg_r = _sc_degrees(s2d, r2d)

    for step in range(2):
        w0 = jnp.kron(eye32, W[step, 0, :, :4].astype(f32))
        w1 = jnp.kron(eye32, W[step, 1, :, :4].astype(f32))
        pp = jnp.tile(jnp.concatenate([b[step, :, :4],
                                       theta[step].reshape(4, 4),
                                       jnp.zeros((2, 4), f32)],
                                      axis=0).astype(f32), (1, 32))
        lp = jnp.tile(jnp.concatenate([ln_scale[step][None, :],
                                       ln_bias[step][None, :],
                                       jnp.zeros((6, 4), f32)],
                                      axis=0).astype(f32), (1, 32))
        table = _tc_table()(nodes, deg_s[0], deg_s[1], w0, w1, pp)
        partials = _sc_messages(s2d, r2d, table)
        nodes = _tc_combine()(partials[0], partials[1], table, nodes,
                              deg_r[0], deg_r[1], mk, lp)

    pool = _tc_pool()(nodes)
    pooled = pool[0:1, :].reshape(32, 4).sum(axis=0, keepdims=True) / float(N)
    globals_out = pooled @ Wd + bd[None, :]

    e128 = (edges.reshape(E // 128, 128, D)
            .transpose(0, 2, 1).reshape(EROWS2, 128))
    o128 = _tc_edges()(e128)
    edges_out = (o128.reshape(E // 128, D, 128)
                 .transpose(0, 2, 1).reshape(E, D))
    nodes_out = nodes.reshape(NPAD, D)[:N]
    return nodes_out, edges_out, globals_out
